# baseline (speedup 1.0000x reference)
import functools

import jax
import jax.numpy as jnp
from jax import lax
from jax.experimental import pallas as pl
from jax.experimental.pallas import tpu as pltpu
from jax.experimental.pallas import tpu_sc as plsc

D_MODEL = 32
CHUNK = 128
NBUF = 4
L = 16


def _winfo():
    info = plsc.get_sparse_core_info()
    return info.num_cores, info.num_subcores


@functools.cache
def _build_transpose(n_rows):
    NC, NS = _winfo()
    NW = NC * NS
    n_full = n_rows // CHUNK
    tail = n_rows - n_full * CHUNK
    per_w = n_full // NW
    extra = n_full - per_w * NW
    max_n = per_w + (1 if extra else 0)
    mesh = plsc.VectorSubcoreMesh(core_axis_name="c", subcore_axis_name="s")

    @functools.partial(
        pl.kernel,
        mesh=mesh,
        compiler_params=pltpu.CompilerParams(
            use_tc_tiling_on_sc=True, needs_layout_passes=False),
        out_type=jax.ShapeDtypeStruct((n_rows * D_MODEL,), jnp.float32),
        scratch_types=[
            [pltpu.VMEM((D_MODEL, CHUNK + 9), jnp.float32) for _ in range(4)],
            [pltpu.VMEM((D_MODEL * CHUNK,), jnp.float32) for _ in range(4)],
            [pltpu.SemaphoreType.DMA for _ in range(4)],
            [pltpu.SemaphoreType.DMA for _ in range(4)],
        ],
    )
    def t_kernel(wt_hbm, tail_hbm, out_hbm, ibufs, obufs, isems, osems):
        wid = lax.axis_index("s") * NC + lax.axis_index("c")
        my_n = jnp.where(wid < extra, per_w + 1, per_w)
        my_base = wid * per_w + jnp.minimum(wid, extra)

        iotas = [lax.iota(jnp.int32, L) + g * L for g in range(2)]

        def fire_read(c, p):
            off = pl.multiple_of(c * CHUNK, CHUNK)
            pltpu.async_copy(
                wt_hbm.at[:, pl.ds(off, CHUNK)],
                ibufs[p].at[:, pl.ds(0, CHUNK)], isems[p])

        def wait_read(p):
            pltpu.make_async_copy(
                wt_hbm.at[:, pl.ds(0, CHUNK)],
                ibufs[p].at[:, pl.ds(0, CHUNK)], isems[p]).wait()

        def wait_write(p):
            pltpu.make_async_copy(
                obufs[p], out_hbm.at[pl.ds(0, D_MODEL * CHUNK)],
                osems[p]).wait()

        def transpose_block(p, nr):
            ib, ob = ibufs[p], obufs[p]
            for r in range(nr):
                for q in range(4):
                    col = jnp.full((L,), 4 * r + q, jnp.int32)
                    for g in range(2):
                        ob[pl.ds(r * CHUNK + q * D_MODEL + g * L, L)] = (
                            plsc.load_gather(ib, [iotas[g], col]))

        def fire_write(c, p):
            pltpu.async_copy(
                obufs[p],
                out_hbm.at[pl.ds(c * (CHUNK * D_MODEL), CHUNK * D_MODEL)],
                osems[p])

        for p in range(4):
            @pl.when(p < my_n)
            def _():
                fire_read(my_base + p, p)

        def body(i, carry):
            for p in range(4):
                s = i * 4 + p

                @pl.when(s < my_n)
                def _():
                    wait_read(p)

                    @pl.when(s >= 4)
                    def _():
                        wait_write(p)

                    transpose_block(p, D_MODEL)

                    @pl.when(s + 4 < my_n)
                    def _():
                        fire_read(my_base + s + 4, p)

                    fire_write(my_base + s, p)
            return carry

        lax.fori_loop(0, (max_n + 3) // 4, body, 0)
        for p in range(4):
            wait_write(p)

        if tail:
            @pl.when(wid == NW - 1)
            def _():
                pltpu.sync_copy(
                    tail_hbm,
                    out_hbm.at[pl.ds(n_full * CHUNK * D_MODEL,
                                     tail * D_MODEL)])

    return t_kernel


@functools.cache
def _build_gather(batch, hist, n_rows):
    NC, NS = _winfo()
    NW = NC * NS
    n_bb = batch // CHUNK
    t_per_w = hist * n_bb // NW
    assert t_per_w % NBUF == 0
    mesh = plsc.VectorSubcoreMesh(core_axis_name="c", subcore_axis_name="s")

    @functools.partial(
        pl.kernel,
        mesh=mesh,
        compiler_params=pltpu.CompilerParams(
            use_tc_tiling_on_sc=False, needs_layout_passes=False),
        out_type=jax.ShapeDtypeStruct((hist, D_MODEL, batch), jnp.float32),
        scratch_types=[
            pltpu.VMEM((t_per_w, CHUNK), jnp.int32),
            [pltpu.VMEM((CHUNK, D_MODEL), jnp.float32) for _ in range(NBUF)],
            [pltpu.VMEM((D_MODEL, CHUNK + 1), jnp.float32)
             for _ in range(NBUF)],
            [pltpu.SemaphoreType.DMA for _ in range(NBUF)],
            [pltpu.SemaphoreType.DMA for _ in range(NBUF)],
        ],
    )
    def g_kernel(idx_hbm, table_hbm, out_hbm, idx_v, rbufs, tbufs,
                 gsems, wsems):
        wid = lax.axis_index("s") * NC + lax.axis_index("c")
        base_t = wid * t_per_w
        pltpu.sync_copy(idx_hbm.at[wid], idx_v)

        iotas = [lax.iota(jnp.int32, L) + kb * L for kb in range(CHUNK // L)]

        def fire_gather(t, p):
            pltpu.async_copy(table_hbm.at[idx_v.at[t]], rbufs[p], gsems[p])

        def transpose(p):
            r, tb = rbufs[p], tbufs[p]
            for k in range(CHUNK):
                colk = jnp.full((L,), k, jnp.int32)
                for m in range(D_MODEL // L):
                    plsc.store_scatter(
                        tb, [iotas[m], colk], r[k, pl.ds(m * L, L)])

        def fire_write(t, p):
            c = base_t + t
            h = c // n_bb
            bb = c % n_bb
            off = pl.multiple_of(bb * CHUNK, CHUNK)
            pltpu.async_copy(
                tbufs[p].at[:, pl.ds(0, CHUNK)],
                out_hbm.at[h, :, pl.ds(off, CHUNK)], wsems[p])

        def wait_gather(p):
            pltpu.make_async_copy(
                table_hbm.at[pl.ds(0, CHUNK)], rbufs[p], gsems[p]).wait()

        def wait_write(p):
            pltpu.make_async_copy(
                tbufs[p].at[:, pl.ds(0, CHUNK)],
                out_hbm.at[0, :, pl.ds(0, CHUNK)], wsems[p]).wait()

        for p in range(NBUF):
            fire_gather(p, p)

        def body(i, carry):
            for p in range(NBUF):
                t = i * NBUF + p
                wait_gather(p)

                @pl.when(t >= NBUF)
                def _():
                    wait_write(p)

                transpose(p)

                @pl.when(t + NBUF < t_per_w)
                def _():
                    fire_gather(t + NBUF, p)

                fire_write(t, p)
            return carry

        lax.fori_loop(0, t_per_w // NBUF, body, 0)
        for p in range(NBUF):
            wait_write(p)

    return g_kernel


def kernel(x, spa_emb_weight):
    batch, hist = x.shape
    n_rows, d = spa_emb_weight.shape
    NC, NS = _winfo()
    NW = NC * NS
    n_idx_per_w = batch * hist // NW
    idx3 = x.T.astype(jnp.int32).reshape(NW, n_idx_per_w // CHUNK, CHUNK)
    wt = spa_emb_weight.T

    n_full = n_rows // CHUNK
    tail_flat = spa_emb_weight[n_full * CHUNK:].reshape(-1)
    flat = _build_transpose(n_rows)(wt, tail_flat)
    table_rm = flat.reshape(n_rows, d)

    outT = _build_gather(batch, hist, n_rows)(idx3, table_rm)
    return jnp.transpose(outT, (2, 0, 1))


# speedup vs baseline: 1.3429x; 1.3429x over previous
import functools

import jax
import jax.numpy as jnp
from jax import lax
from jax.experimental import pallas as pl
from jax.experimental.pallas import tpu as pltpu
from jax.experimental.pallas import tpu_sc as plsc

D_MODEL = 32
CHUNK = 128
NBUF = 4
L = 16


def _winfo():
    info = plsc.get_sparse_core_info()
    return info.num_cores, info.num_subcores


@functools.cache
def _build_transpose(n_rows):
    NC, NS = _winfo()
    NW = NC * NS
    n_full = n_rows // CHUNK
    tail = n_rows - n_full * CHUNK
    per_w = n_full // NW
    extra = n_full - per_w * NW
    max_n = per_w + (1 if extra else 0)
    mesh = plsc.VectorSubcoreMesh(core_axis_name="c", subcore_axis_name="s")

    @functools.partial(
        pl.kernel,
        mesh=mesh,
        compiler_params=pltpu.CompilerParams(
            use_tc_tiling_on_sc=True, needs_layout_passes=False),
        out_type=jax.ShapeDtypeStruct(
            (n_rows * D_MODEL // CHUNK, CHUNK), jnp.float32),
        scratch_types=[
            [pltpu.VMEM((D_MODEL, CHUNK), jnp.float32) for _ in range(4)],
            [pltpu.VMEM((D_MODEL, CHUNK + 1), jnp.float32)
             for _ in range(4)],
            [pltpu.SemaphoreType.DMA for _ in range(4)],
            [pltpu.SemaphoreType.DMA for _ in range(4)],
        ],
    )
    def t_kernel(wt_hbm, tail_hbm, out_hbm, ibufs, obufs, isems, osems):
        wid = lax.axis_index("s") * NC + lax.axis_index("c")
        my_n = jnp.where(wid < extra, per_w + 1, per_w)
        my_base = wid * per_w + jnp.minimum(wid, extra)

        iotas = [lax.iota(jnp.int32, L) + g * L for g in range(CHUNK // L)]

        def fire_read(c, p):
            off = pl.multiple_of(c * CHUNK, CHUNK)
            pltpu.async_copy(
                wt_hbm.at[:, pl.ds(off, CHUNK)], ibufs[p], isems[p])

        def wait_read(p):
            pltpu.make_async_copy(
                wt_hbm.at[:, pl.ds(0, CHUNK)], ibufs[p], isems[p]).wait()

        def wait_write(p):
            pltpu.make_async_copy(
                obufs[p].at[:, pl.ds(0, CHUNK)],
                out_hbm.at[pl.ds(0, D_MODEL), :], osems[p]).wait()

        rvecs = [(lax.iota(jnp.int32, L) + g * L) // 4
                 for g in range(CHUNK // L)]
        cvecs = [((lax.iota(jnp.int32, L) + g * L) % 4) * D_MODEL
                 for g in range(CHUNK // L)]

        def transpose_block(p):
            ib, ob = ibufs[p], obufs[p]
            for j in range(D_MODEL):
                colj = jnp.full((L,), j, jnp.int32)
                for m in range(CHUNK // L):
                    plsc.store_scatter(
                        ob, [rvecs[m], cvecs[m] + colj],
                        ib[j, pl.ds(m * L, L)])

        def fire_write(c, p):
            off = pl.multiple_of(c * D_MODEL, D_MODEL)
            pltpu.async_copy(
                obufs[p].at[:, pl.ds(0, CHUNK)],
                out_hbm.at[pl.ds(off, D_MODEL), :], osems[p])

        for p in range(4):
            @pl.when(p < my_n)
            def _():
                fire_read(my_base + p, p)

        def body(i, carry):
            for p in range(4):
                s = i * 4 + p

                @pl.when(s < my_n)
                def _():
                    wait_read(p)

                    @pl.when(s >= 4)
                    def _():
                        wait_write(p)

                    transpose_block(p)

                    @pl.when(s + 4 < my_n)
                    def _():
                        fire_read(my_base + s + 4, p)

                    fire_write(my_base + s, p)
            return carry

        lax.fori_loop(0, (max_n + 3) // 4, body, 0)
        for p in range(4):
            wait_write(p)

        if tail:
            @pl.when(wid == NW - 1)
            def _():
                pltpu.sync_copy(
                    tail_hbm,
                    out_hbm.at[pl.ds(n_full * D_MODEL,
                                     tail * D_MODEL // CHUNK), :])

    return t_kernel


@functools.cache
def _build_gather(batch, hist, n_rows):
    NC, NS = _winfo()
    NW = NC * NS
    n_bb = batch // CHUNK
    t_per_w = hist * n_bb // NW
    assert t_per_w % NBUF == 0
    mesh = plsc.VectorSubcoreMesh(core_axis_name="c", subcore_axis_name="s")

    @functools.partial(
        pl.kernel,
        mesh=mesh,
        compiler_params=pltpu.CompilerParams(
            use_tc_tiling_on_sc=False, needs_layout_passes=False),
        out_type=jax.ShapeDtypeStruct((hist, D_MODEL, batch), jnp.float32),
        scratch_types=[
            pltpu.VMEM((t_per_w, CHUNK), jnp.int32),
            [pltpu.VMEM((CHUNK, D_MODEL), jnp.float32) for _ in range(NBUF)],
            [pltpu.VMEM((D_MODEL, CHUNK + 1), jnp.float32)
             for _ in range(NBUF)],
            [pltpu.SemaphoreType.DMA for _ in range(NBUF)],
            [pltpu.SemaphoreType.DMA for _ in range(NBUF)],
        ],
    )
    def g_kernel(idx_hbm, table_hbm, out_hbm, idx_v, rbufs, tbufs,
                 gsems, wsems):
        wid = lax.axis_index("s") * NC + lax.axis_index("c")
        base_t = wid * t_per_w
        pltpu.sync_copy(idx_hbm.at[wid], idx_v)

        iotas = [lax.iota(jnp.int32, L) + kb * L for kb in range(CHUNK // L)]

        def fire_gather(t, p):
            pltpu.async_copy(table_hbm.at[idx_v.at[t]], rbufs[p], gsems[p])

        def transpose(p):
            r, tb = rbufs[p], tbufs[p]
            for k in range(CHUNK):
                colk = jnp.full((L,), k, jnp.int32)
                for m in range(D_MODEL // L):
                    plsc.store_scatter(
                        tb, [iotas[m], colk], r[k, pl.ds(m * L, L)])

        def fire_write(t, p):
            c = base_t + t
            h = c // n_bb
            bb = c % n_bb
            off = pl.multiple_of(bb * CHUNK, CHUNK)
            pltpu.async_copy(
                tbufs[p].at[:, pl.ds(0, CHUNK)],
                out_hbm.at[h, :, pl.ds(off, CHUNK)], wsems[p])

        def wait_gather(p):
            pltpu.make_async_copy(
                table_hbm.at[pl.ds(0, CHUNK)], rbufs[p], gsems[p]).wait()

        def wait_write(p):
            pltpu.make_async_copy(
                tbufs[p].at[:, pl.ds(0, CHUNK)],
                out_hbm.at[0, :, pl.ds(0, CHUNK)], wsems[p]).wait()

        for p in range(NBUF):
            fire_gather(p, p)

        def body(i, carry):
            for p in range(NBUF):
                t = i * NBUF + p
                wait_gather(p)

                @pl.when(t >= NBUF)
                def _():
                    wait_write(p)

                transpose(p)

                @pl.when(t + NBUF < t_per_w)
                def _():
                    fire_gather(t + NBUF, p)

                fire_write(t, p)
            return carry

        lax.fori_loop(0, t_per_w // NBUF, body, 0)
        for p in range(NBUF):
            wait_write(p)

    return g_kernel


def kernel(x, spa_emb_weight):
    batch, hist = x.shape
    n_rows, d = spa_emb_weight.shape
    NC, NS = _winfo()
    NW = NC * NS
    n_idx_per_w = batch * hist // NW
    idx3 = x.T.astype(jnp.int32).reshape(NW, n_idx_per_w // CHUNK, CHUNK)
    wt = spa_emb_weight.T

    n_full = n_rows // CHUNK
    tail_packed = spa_emb_weight[n_full * CHUNK:].reshape(-1, CHUNK)
    packed = _build_transpose(n_rows)(wt, tail_packed)
    table_rm = packed.reshape(n_rows, d)

    outT = _build_gather(batch, hist, n_rows)(idx3, table_rm)
    return jnp.transpose(outT, (2, 0, 1))
